# baseline (device time: 45047 ns/iter reference)
import jax
import jax.numpy as jnp
from jax import lax
from jax.experimental import pallas as pl
from jax.experimental.pallas import tpu as pltpu

N_DEV = 4
B_LOC = 2
SQ = 128
SKV = 128
HQ = 16
H_LOC = 4
DH = 64
D_MODEL = 512
HD_LOC = H_LOC * DH

BF16 = jnp.bfloat16
F32 = jnp.float32


def kernel(x, Wq, K_ext, V_ext, Wo):
    my = lax.axis_index("i")

    k_loc = lax.dynamic_slice_in_dim(K_ext, B_LOC * my, B_LOC, axis=0)
    v_loc = lax.dynamic_slice_in_dim(V_ext, B_LOC * my, B_LOC, axis=0)
    k_loc = jnp.transpose(k_loc, (0, 2, 1, 3)).reshape(B_LOC * HQ, SKV, DH)
    v_loc = jnp.transpose(v_loc, (0, 2, 1, 3)).reshape(B_LOC * HQ, SKV, DH)
    k_loc = k_loc.astype(BF16)
    v_loc = v_loc.astype(BF16)

    def body(x_ref, wq_ref, k_ref, v_ref, wo_ref, out_ref,
             wq_all, wo_all, wq_ssem, wq_rsem, wo_ssem, wo_rsem):
        my_pos = lax.axis_index("i")
        left = lax.rem(my_pos + N_DEV - 1, N_DEV)
        right = lax.rem(my_pos + 1, N_DEV)

        wq_all[pl.ds(my_pos, 1)] = wq_ref[...].astype(BF16)[None]
        wo_all[pl.ds(my_pos, 1)] = wo_ref[...].astype(BF16)[None]

        barrier_sem = pltpu.get_barrier_semaphore()
        for nbr in (left, right):
            pl.semaphore_signal(
                barrier_sem, inc=1,
                device_id=(nbr,), device_id_type=pl.DeviceIdType.MESH,
            )
        pl.semaphore_wait(barrier_sem, 2)

        for h in range(N_DEV - 1):
            src = lax.rem(my_pos + N_DEV - h, N_DEV)
            rdma_q = pltpu.make_async_remote_copy(
                src_ref=wq_all.at[src],
                dst_ref=wq_all.at[src],
                send_sem=wq_ssem.at[h],
                recv_sem=wq_rsem.at[h],
                device_id=(right,),
                device_id_type=pl.DeviceIdType.MESH,
            )
            rdma_o = pltpu.make_async_remote_copy(
                src_ref=wo_all.at[src],
                dst_ref=wo_all.at[src],
                send_sem=wo_ssem.at[h],
                recv_sem=wo_rsem.at[h],
                device_id=(right,),
                device_id_type=pl.DeviceIdType.MESH,
            )
            rdma_q.start()
            rdma_o.start()
            rdma_q.wait()
            rdma_o.wait()

        row = lax.broadcasted_iota(jnp.int32, (SQ, SKV), 0)
        col = lax.broadcasted_iota(jnp.int32, (SQ, SKV), 1)
        keep = (col // 64) <= (row // 64)

        for b in range(B_LOC):
            xb = x_ref[b].astype(BF16)
            acc = jnp.zeros((SQ, D_MODEL), F32)
            for j in range(N_DEV):
                q_blk = lax.dot_general(
                    xb, wq_all[j],
                    (((1,), (0,)), ((), ())),
                    preferred_element_type=F32,
                )
                for r in range(H_LOC):
                    h_glob = j * H_LOC + r
                    q = q_blk[:, r * DH:(r + 1) * DH].astype(BF16)
                    k = k_ref[b * HQ + h_glob]
                    v = v_ref[b * HQ + h_glob]
                    s = lax.dot_general(
                        q, k, (((1,), (1,)), ((), ())),
                        preferred_element_type=F32,
                    ) * 0.125
                    s = jnp.where(keep, s, -1e9)
                    m = jnp.max(s, axis=-1, keepdims=True)
                    w = jnp.exp(s - m)
                    w = w / jnp.sum(w, axis=-1, keepdims=True)
                    ctx = lax.dot_general(
                        w.astype(BF16), v, (((1,), (0,)), ((), ())),
                        preferred_element_type=F32,
                    )
                    acc = acc + lax.dot_general(
                        ctx.astype(BF16),
                        wo_all[j][r * DH:(r + 1) * DH, :],
                        (((1,), (0,)), ((), ())),
                        preferred_element_type=F32,
                    )
            out_ref[b] = acc

    return pl.pallas_call(
        body,
        out_shape=jax.ShapeDtypeStruct((B_LOC, SQ, D_MODEL), F32),
        in_specs=[
            pl.BlockSpec(memory_space=pltpu.VMEM),
            pl.BlockSpec(memory_space=pltpu.VMEM),
            pl.BlockSpec(memory_space=pltpu.VMEM),
            pl.BlockSpec(memory_space=pltpu.VMEM),
            pl.BlockSpec(memory_space=pltpu.VMEM),
        ],
        out_specs=pl.BlockSpec(memory_space=pltpu.VMEM),
        scratch_shapes=[
            pltpu.VMEM((N_DEV, D_MODEL, HD_LOC), BF16),
            pltpu.VMEM((N_DEV, HD_LOC, D_MODEL), BF16),
            pltpu.SemaphoreType.DMA((N_DEV - 1,)),
            pltpu.SemaphoreType.DMA((N_DEV - 1,)),
            pltpu.SemaphoreType.DMA((N_DEV - 1,)),
            pltpu.SemaphoreType.DMA((N_DEV - 1,)),
        ],
        compiler_params=pltpu.CompilerParams(collective_id=0),
    )(x, Wq, k_loc, v_loc, Wo)


# device time: 20429 ns/iter; 2.2051x vs baseline; 2.2051x over previous
import jax
import jax.numpy as jnp
from jax import lax
from jax.experimental import pallas as pl
from jax.experimental.pallas import tpu as pltpu

N_DEV = 4
B_LOC = 2
SQ = 128
SKV = 128
HQ = 16
H_LOC = 4
DH = 64
D_MODEL = 512
HD_LOC = H_LOC * DH

BF16 = jnp.bfloat16
F32 = jnp.float32


def kernel(x, Wq, K_ext, V_ext, Wo):
    my = lax.axis_index("i")

    k_loc = lax.dynamic_slice_in_dim(K_ext, B_LOC * my, B_LOC, axis=0)
    v_loc = lax.dynamic_slice_in_dim(V_ext, B_LOC * my, B_LOC, axis=0)
    k_loc = jnp.transpose(k_loc, (0, 2, 1, 3)).reshape(B_LOC * HQ, SKV, DH)
    v_loc = jnp.transpose(v_loc, (0, 2, 1, 3)).reshape(B_LOC * HQ, SKV, DH)
    k_loc = k_loc.astype(BF16)
    v_loc = v_loc.astype(BF16)

    def body(x_ref, wq_ref, k_ref, v_ref, wo_ref, out_ref,
             wq_all, wo_all, wq_ssem, wq_rsem, wo_ssem, wo_rsem):
        my_pos = lax.axis_index("i")
        left = lax.rem(my_pos + N_DEV - 1, N_DEV)
        right = lax.rem(my_pos + 1, N_DEV)
        jm1 = left
        jp1 = right
        jm2 = lax.rem(my_pos + 2, N_DEV)

        wq_all[pl.ds(my_pos, 1)] = wq_ref[...].astype(BF16)[None]
        wo_all[pl.ds(my_pos, 1)] = wo_ref[...].astype(BF16)[None]

        barrier_sem = pltpu.get_barrier_semaphore()
        for nbr in (left, right):
            pl.semaphore_signal(
                barrier_sem, inc=1,
                device_id=(nbr,), device_id_type=pl.DeviceIdType.MESH,
            )
        pl.semaphore_wait(barrier_sem, 2)

        def copy(buf, src_slot, dst_slot, ssem, rsem, slot, dev):
            return pltpu.make_async_remote_copy(
                src_ref=buf.at[src_slot],
                dst_ref=buf.at[dst_slot],
                send_sem=ssem.at[slot],
                recv_sem=rsem.at[slot],
                device_id=(dev,),
                device_id_type=pl.DeviceIdType.MESH,
            )

        q_r0 = copy(wq_all, my_pos, my_pos, wq_ssem, wq_rsem, 0, right)
        q_l0 = copy(wq_all, my_pos, my_pos, wq_ssem, wq_rsem, 1, left)
        o_r0 = copy(wo_all, my_pos, my_pos, wo_ssem, wo_rsem, 0, right)
        o_l0 = copy(wo_all, my_pos, my_pos, wo_ssem, wo_rsem, 1, left)
        q_r0.start()
        q_l0.start()
        o_r0.start()
        o_l0.start()

        q_recv0 = copy(wq_all, jm1, jm1, wq_ssem, wq_rsem, 0, left)
        q_recv1 = copy(wq_all, jp1, jp1, wq_ssem, wq_rsem, 1, right)
        q_recv2 = copy(wq_all, jm2, jm2, wq_ssem, wq_rsem, 2, left)
        o_recv0 = copy(wo_all, jm1, jm1, wo_ssem, wo_rsem, 0, left)
        o_recv1 = copy(wo_all, jp1, jp1, wo_ssem, wo_rsem, 1, right)
        o_recv2 = copy(wo_all, jm2, jm2, wo_ssem, wo_rsem, 2, right)

        row = lax.broadcasted_iota(jnp.int32, (SQ, SKV), 0)
        col = lax.broadcasted_iota(jnp.int32, (SQ, SKV), 1)
        keep = (col // 64) <= (row // 64)

        xs = [(x_ref[b] * 0.125).astype(BF16) for b in range(B_LOC)]

        def block_contrib(j, accs):
            wq_j = wq_all[pl.ds(j, 1)].reshape(D_MODEL, HD_LOC)
            wo_j = wo_all[pl.ds(j, 1)].reshape(HD_LOC, D_MODEL)
            out = []
            for b in range(B_LOC):
                q_blk = lax.dot_general(
                    xs[b], wq_j, (((1,), (0,)), ((), ())),
                    preferred_element_type=F32,
                ).astype(BF16)
                ctxs = []
                for r in range(H_LOC):
                    h_idx = b * HQ + j * H_LOC + r
                    q = q_blk[:, r * DH:(r + 1) * DH]
                    k = k_ref[pl.ds(h_idx, 1)].reshape(SKV, DH)
                    v = v_ref[pl.ds(h_idx, 1)].reshape(SKV, DH)
                    s = lax.dot_general(
                        q, k, (((1,), (1,)), ((), ())),
                        preferred_element_type=F32,
                    )
                    e = jnp.where(keep, jnp.exp(s), 0.0)
                    rs = 1.0 / jnp.sum(e, axis=-1, keepdims=True)
                    ctx = lax.dot_general(
                        e.astype(BF16), v, (((1,), (0,)), ((), ())),
                        preferred_element_type=F32,
                    )
                    ctxs.append((ctx * rs).astype(BF16))
                ctx_cat = jnp.concatenate(ctxs, axis=1)
                out.append(accs[b] + lax.dot_general(
                    ctx_cat, wo_j, (((1,), (0,)), ((), ())),
                    preferred_element_type=F32,
                ))
            return out

        accs = [jnp.zeros((SQ, D_MODEL), F32) for _ in range(B_LOC)]

        accs = block_contrib(my_pos, accs)

        q_recv0.wait_recv()
        o_recv0.wait_recv()
        q_f = copy(wq_all, jm1, jm1, wq_ssem, wq_rsem, 2, right)
        q_f.start()
        o_recv1.wait_recv()
        o_f = copy(wo_all, jp1, jp1, wo_ssem, wo_rsem, 2, left)
        o_f.start()
        accs = block_contrib(jm1, accs)

        q_recv1.wait_recv()
        accs = block_contrib(jp1, accs)

        q_recv2.wait_recv()
        o_recv2.wait_recv()
        accs = block_contrib(jm2, accs)

        for b in range(B_LOC):
            out_ref[b] = accs[b]

        for d in (q_r0, q_l0, o_r0, o_l0, q_f, o_f):
            d.wait_send()

    return pl.pallas_call(
        body,
        out_shape=jax.ShapeDtypeStruct((B_LOC, SQ, D_MODEL), F32),
        in_specs=[
            pl.BlockSpec(memory_space=pltpu.VMEM),
            pl.BlockSpec(memory_space=pltpu.VMEM),
            pl.BlockSpec(memory_space=pltpu.VMEM),
            pl.BlockSpec(memory_space=pltpu.VMEM),
            pl.BlockSpec(memory_space=pltpu.VMEM),
        ],
        out_specs=pl.BlockSpec(memory_space=pltpu.VMEM),
        scratch_shapes=[
            pltpu.VMEM((N_DEV, D_MODEL, HD_LOC), BF16),
            pltpu.VMEM((N_DEV, HD_LOC, D_MODEL), BF16),
            pltpu.SemaphoreType.DMA((3,)),
            pltpu.SemaphoreType.DMA((3,)),
            pltpu.SemaphoreType.DMA((3,)),
            pltpu.SemaphoreType.DMA((3,)),
        ],
        compiler_params=pltpu.CompilerParams(collective_id=0),
    )(x, Wq, k_loc, v_loc, Wo)
